# Initial kernel scaffold; baseline (speedup 1.0000x reference)
import jax
import jax.numpy as jnp
from jax.experimental import pallas as pl


def kernel(x, l, dom_table, W, b, cls):
    raise NotImplementedError("write your pallas kernel here")


# speedup vs baseline: 2.3851x; 2.3851x over previous
import functools

import jax
import jax.numpy as jnp
from jax import lax
from jax.experimental import pallas as pl
from jax.experimental.pallas import tpu as pltpu
from jax.experimental.pallas import tpu_sc as plsc

B = 4096
S = 200
R = S + 1
NT = B * R
NW = 32
PW = NT // NW
CHUNK = 128
NCHUNK = PW // CHUNK
D = 64
CLS_ROW = 5162


def _sc_gather(table, src):
    mesh = plsc.VectorSubcoreMesh(core_axis_name="c", subcore_axis_name="s")

    @functools.partial(
        pl.kernel,
        out_type=jax.ShapeDtypeStruct((NT, D), jnp.float32),
        mesh=mesh,
        scratch_types=[
            pltpu.VMEM((CHUNK,), jnp.int32),
            pltpu.VMEM((CHUNK, D), jnp.float32),
            pltpu.SemaphoreType.DMA,
        ],
        compiler_params=pltpu.CompilerParams(use_tc_tiling_on_sc=False),
    )
    def sc_kernel(table_hbm, src_hbm, out_hbm, src_v, rows_v, g_sem):
        wid = lax.axis_index("s") * 2 + lax.axis_index("c")
        base = wid * PW

        @pl.loop(0, NCHUNK)
        def _(c):
            start = base + c * CHUNK
            pltpu.sync_copy(src_hbm.at[pl.ds(start, CHUNK)], src_v)
            pltpu.async_copy(table_hbm.at[src_v], rows_v, g_sem).wait()
            pltpu.sync_copy(rows_v, out_hbm.at[pl.ds(start, CHUNK)])

    return sc_kernel(table, src)


BB = 16


def _tc_body(y_ref, x_ref, l_ref, wt_ref, b_ref, clshi_ref, emb_ref, mask_ref):
    x0 = x_ref[:, :, 0:1]
    x1 = x_ref[:, :, 1:2]
    x2 = x_ref[:, :, 2:3]
    dense = (x0 * wt_ref[0:1, :] + x1 * wt_ref[1:2, :]
             + x2 * wt_ref[2:3, :] + b_ref[0:1, :])
    hi = jnp.concatenate(
        [jnp.broadcast_to(clshi_ref[...], (BB, 1, D)), dense], axis=1)
    emb_ref[...] = jnp.concatenate([y_ref[...], hi], axis=2)
    pos = lax.broadcasted_iota(jnp.int32, (BB, R), 1)
    mask_ref[...] = (pos >= l_ref[...] + 1).astype(jnp.int8)


def _tc_pass(y, x, l2, wt, b2, clshi):
    return pl.pallas_call(
        _tc_body,
        grid=(B // BB,),
        in_specs=[
            pl.BlockSpec((BB, R, D), lambda i: (i, 0, 0)),
            pl.BlockSpec((BB, S, 4), lambda i: (i, 0, 0)),
            pl.BlockSpec((BB, 1), lambda i: (i, 0)),
            pl.BlockSpec((3, D), lambda i: (0, 0)),
            pl.BlockSpec((1, D), lambda i: (0, 0)),
            pl.BlockSpec((1, 1, D), lambda i: (0, 0, 0)),
        ],
        out_specs=[
            pl.BlockSpec((BB, R, 2 * D), lambda i: (i, 0, 0)),
            pl.BlockSpec((BB, R), lambda i: (i, 0)),
        ],
        out_shape=[
            jax.ShapeDtypeStruct((B, R, 2 * D), jnp.float32),
            jax.ShapeDtypeStruct((B, R), jnp.int8),
        ],
    )(y, x, l2, wt, b2, clshi)


def kernel(x, l, dom_table, W, b, cls):
    dom_idx = x[:, :, 3].astype(jnp.int32)
    src = jnp.concatenate(
        [jnp.full((B, 1), CLS_ROW, jnp.int32), dom_idx], axis=1
    ).reshape(NT)
    table = jnp.concatenate([dom_table, cls[0, :, :D]], axis=0)
    y = _sc_gather(table, src).reshape(B, R, D)
    emb, mask8 = _tc_pass(y, x, l.reshape(B, 1), W.T, b.reshape(1, D),
                          cls[:, :, D:])
    return emb, mask8.astype(jnp.bool_)
